# baseline (speedup 1.0000x reference)
import functools

import jax
import jax.numpy as jnp
from jax.experimental import pallas as pl
from jax.experimental.pallas import tpu as pltpu

BN_EPS = 1e-5
LANE = 128
SUBLANE = 8
VMEM_LIMIT = 60 * 1024 * 1024


def _fused_kernel(x_ref, w_ref, gamma_ref, beta_ref, o_ref,
                  xc_ref, acc_ref, ab_ref, *,
                  bn, n_chunks, pad, ho, wo, inv_cnt):
    p = pl.program_id(0)
    b = pl.program_id(1)

    @pl.when(p == 0)
    def _phase0():
        @pl.when(b == 0)
        def _():
            acc_ref[...] = jnp.zeros_like(acc_ref)

        xc = x_ref[:, 0, pad:pad + ho, pad:pad + wo]
        xc_ref[pl.ds(b * bn, bn)] = xc
        v = xc.reshape(-1, SUBLANE, LANE)
        acc_ref[0] += jnp.sum(v, axis=0)
        acc_ref[1] += jnp.sum(v * v, axis=0)

        @pl.when(b == n_chunks - 1)
        def _finalize():
            s1 = jnp.sum(acc_ref[0])
            s2 = jnp.sum(acc_ref[1])
            w = w_ref[0]
            mean_t = w * s1 * inv_cnt
            ex2_t = w * w * s2 * inv_cnt
            var = jnp.maximum(ex2_t - mean_t * mean_t, 0.0)
            a = gamma_ref[0] * jax.lax.rsqrt(var + BN_EPS)
            ab_ref[0] = w * a
            ab_ref[1] = beta_ref[0] - mean_t * a

    @pl.when(p == 1)
    def _phase1():
        a = ab_ref[0]
        c = ab_ref[1]
        xc = xc_ref[pl.ds(b * bn, bn)]
        o_ref[:, 0] = jnp.maximum(xc * a + c, 0.0)


@functools.partial(jax.jit, static_argnames=("stride", "padding"))
def _forward(x, w_t, gamma, beta, *, stride=1, padding=1):
    N, Cin, H, W = x.shape
    Cin_w, Cout, kH, kW = w_t.shape
    assert Cin == 1 and Cout == 1 and kH == 1 and kW == 1 and stride == 1

    Ho = (H - 1) * stride - 2 * padding + kH
    Wo = (W - 1) * stride - 2 * padding + kW
    assert Ho > 0 and Wo > 0

    bn = 1
    for cand in (32, 16, 8, 4, 2):
        if N % cand == 0 and (cand * Ho * Wo) % (SUBLANE * LANE) == 0:
            bn = cand
            break
    assert (bn * Ho * Wo) % (SUBLANE * LANE) == 0
    n_chunks = N // bn

    w1 = w_t.reshape(1).astype(jnp.float32)
    gamma32 = gamma.astype(jnp.float32)
    beta32 = beta.astype(jnp.float32)
    inv_cnt = 1.0 / float(N * Ho * Wo)

    out = pl.pallas_call(
        functools.partial(_fused_kernel, bn=bn, n_chunks=n_chunks,
                          pad=padding, ho=Ho, wo=Wo, inv_cnt=inv_cnt),
        out_shape=jax.ShapeDtypeStruct((N, Cout, Ho, Wo), x.dtype),
        grid=(2, n_chunks),
        in_specs=[
            pl.BlockSpec((bn, 1, H, W),
                         lambda p, b: (b * (1 - p) + (n_chunks - 1) * p, 0, 0, 0)),
            pl.BlockSpec(memory_space=pltpu.MemorySpace.SMEM),
            pl.BlockSpec(memory_space=pltpu.MemorySpace.SMEM),
            pl.BlockSpec(memory_space=pltpu.MemorySpace.SMEM),
        ],
        out_specs=pl.BlockSpec((bn, 1, Ho, Wo), lambda p, b: (b * p, 0, 0, 0)),
        scratch_shapes=[
            pltpu.VMEM((N, Ho, Wo), jnp.float32),
            pltpu.VMEM((2, SUBLANE, LANE), jnp.float32),
            pltpu.SMEM((2,), jnp.float32),
        ],
        compiler_params=pltpu.CompilerParams(
            dimension_semantics=("arbitrary", "arbitrary"),
            vmem_limit_bytes=VMEM_LIMIT),
    )(x, w1, gamma32, beta32)

    return out


def kernel(x, w_t, gamma, beta):
    return _forward(x, w_t, gamma, beta, stride=1, padding=1)


# speedup vs baseline: 1.4751x; 1.4751x over previous
import functools

import jax
import jax.numpy as jnp
from jax.experimental import pallas as pl
from jax.experimental.pallas import tpu as pltpu

BN_EPS = 1e-5
LANE = 128
SUBLANE = 8
VMEM_LIMIT = 60 * 1024 * 1024


def _fused_kernel(x_ref, w_ref, gamma_ref, beta_ref, o_ref,
                  xc_ref, acc_ref, ab_ref, *,
                  spb, nb0, nb1, rows_per_slab, crop_rows, pad_rows,
                  h_lo, h_hi, out_rows_per_step, inv_cnt):
    p = pl.program_id(0)
    b = pl.program_id(1)

    @pl.when((p == 0) & (b < nb0))
    def _phase0():
        @pl.when(b == 0)
        def _():
            acc_ref[...] = jnp.zeros_like(acc_ref)

        v = x_ref[...].reshape(spb, rows_per_slab, LANE)
        vc = v[:, pad_rows:pad_rows + crop_rows, :]
        xc_ref[pl.ds(b * spb * crop_rows, spb * crop_rows)] = (
            vc.reshape(spb * crop_rows, LANE))

        gh = b * spb + jax.lax.broadcasted_iota(jnp.int32, (spb, 1, 1), 0)
        mask = ((gh >= h_lo) & (gh < h_hi)).astype(jnp.float32)
        vm = vc * mask
        vm8 = vm.reshape(-1, SUBLANE, LANE)
        acc_ref[0] += jnp.sum(vm8, axis=0)
        acc_ref[1] += jnp.sum(vm8 * vm8, axis=0)

        @pl.when(b == nb0 - 1)
        def _finalize():
            s1 = jnp.sum(acc_ref[0])
            s2 = jnp.sum(acc_ref[1])
            w = w_ref[0]
            mean_t = w * s1 * inv_cnt
            ex2_t = w * w * s2 * inv_cnt
            var = jnp.maximum(ex2_t - mean_t * mean_t, 0.0)
            a = gamma_ref[0] * jax.lax.rsqrt(var + BN_EPS)
            ab_ref[0] = w * a
            ab_ref[1] = beta_ref[0] - mean_t * a

    @pl.when((p == 1) & (b < nb1))
    def _phase1():
        a = ab_ref[0]
        c = ab_ref[1]
        base = h_lo * crop_rows + b * out_rows_per_step
        xc = xc_ref[pl.ds(base, out_rows_per_step)]
        o_ref[...] = jnp.maximum(xc * a + c, 0.0)


@functools.partial(jax.jit, static_argnames=("stride", "padding"))
def _forward(x, w_t, gamma, beta, *, stride=1, padding=1):
    N, Cin, H, W = x.shape
    Cin_w, Cout, kH, kW = w_t.shape
    assert Cin == 1 and Cout == 1 and kH == 1 and kW == 1 and stride == 1

    Ho = (H - 1) * stride - 2 * padding + kH
    Wo = (W - 1) * stride - 2 * padding + kW
    assert Ho > 0 and Wo > 0
    assert N % LANE == 0 and (W * N) % LANE == 0 and (padding * N) % LANE == 0

    rows_per_slab = W * N // LANE
    crop_rows = Wo * N // LANE
    pad_rows = padding * N // LANE

    spb = 1
    for cand in (16, 13, 10, 8, 5, 4, 2):
        if H % cand == 0:
            spb = cand
            break
    nb0 = H // spb

    out_rows = Ho * crop_rows
    nb1 = 16
    while out_rows % nb1 != 0:
        nb1 //= 2
    out_rows_per_step = out_rows // nb1
    nsteps = max(nb0, nb1)

    z = jnp.transpose(x, (2, 3, 1, 0)).reshape(H * rows_per_slab, LANE)
    w1 = w_t.reshape(1).astype(jnp.float32)
    gamma32 = gamma.astype(jnp.float32)
    beta32 = beta.astype(jnp.float32)
    inv_cnt = 1.0 / float(N * Ho * Wo)

    out = pl.pallas_call(
        functools.partial(
            _fused_kernel, spb=spb, nb0=nb0, nb1=nb1,
            rows_per_slab=rows_per_slab,
            crop_rows=crop_rows, pad_rows=pad_rows,
            h_lo=padding, h_hi=padding + Ho,
            out_rows_per_step=out_rows_per_step, inv_cnt=inv_cnt),
        out_shape=jax.ShapeDtypeStruct((out_rows, LANE), x.dtype),
        grid=(2, nsteps),
        in_specs=[
            pl.BlockSpec(
                (spb * rows_per_slab, LANE),
                lambda p, b: (jnp.where(p == 0, jnp.minimum(b, nb0 - 1),
                                        nb0 - 1), 0)),
            pl.BlockSpec(memory_space=pltpu.MemorySpace.SMEM),
            pl.BlockSpec(memory_space=pltpu.MemorySpace.SMEM),
            pl.BlockSpec(memory_space=pltpu.MemorySpace.SMEM),
        ],
        out_specs=pl.BlockSpec((out_rows_per_step, LANE),
                               lambda p, b: (jnp.minimum(b, nb1 - 1) * p, 0)),
        scratch_shapes=[
            pltpu.VMEM((H * crop_rows, LANE), jnp.float32),
            pltpu.VMEM((2, SUBLANE, LANE), jnp.float32),
            pltpu.SMEM((2,), jnp.float32),
        ],
        compiler_params=pltpu.CompilerParams(
            dimension_semantics=("arbitrary", "arbitrary"),
            vmem_limit_bytes=VMEM_LIMIT),
    )(z, w1, gamma32, beta32)

    return out.reshape(Ho, Wo, Cout, N).transpose(3, 2, 0, 1)


def kernel(x, w_t, gamma, beta):
    return _forward(x, w_t, gamma, beta, stride=1, padding=1)


# speedup vs baseline: 2.1766x; 1.4756x over previous
import functools

import jax
import jax.numpy as jnp
from jax.experimental import layout as jax_layout
from jax.experimental import pallas as pl
from jax.experimental.pallas import tpu as pltpu

BN_EPS = 1e-5
LANE = 128
SUBLANE = 8
VMEM_LIMIT = 60 * 1024 * 1024


def _fused_kernel(x_ref, w_ref, gamma_ref, beta_ref, o_ref,
                  xc_ref, acc_ref, ab_ref, *,
                  spb, nb0, nb1, rows_per_slab, crop_rows, pad_rows,
                  h_lo, h_hi, out_rows_per_step, inv_cnt):
    p = pl.program_id(0)
    b = pl.program_id(1)

    @pl.when((p == 0) & (b < nb0))
    def _phase0():
        @pl.when(b == 0)
        def _():
            acc_ref[...] = jnp.zeros_like(acc_ref)

        v = x_ref[...].reshape(spb, rows_per_slab, LANE)
        vc = v[:, pad_rows:pad_rows + crop_rows, :]
        xc_ref[pl.ds(b * spb * crop_rows, spb * crop_rows)] = (
            vc.reshape(spb * crop_rows, LANE))

        gh = b * spb + jax.lax.broadcasted_iota(jnp.int32, (spb, 1, 1), 0)
        mask = ((gh >= h_lo) & (gh < h_hi)).astype(jnp.float32)
        vm = vc * mask
        vm8 = vm.reshape(-1, SUBLANE, LANE)
        acc_ref[0] += jnp.sum(vm8, axis=0)
        acc_ref[1] += jnp.sum(vm8 * vm8, axis=0)

        @pl.when(b == nb0 - 1)
        def _finalize():
            s1 = jnp.sum(acc_ref[0])
            s2 = jnp.sum(acc_ref[1])
            w = w_ref[0]
            mean_t = w * s1 * inv_cnt
            ex2_t = w * w * s2 * inv_cnt
            var = jnp.maximum(ex2_t - mean_t * mean_t, 0.0)
            a = gamma_ref[0] * jax.lax.rsqrt(var + BN_EPS)
            ab_ref[0] = w * a
            ab_ref[1] = beta_ref[0] - mean_t * a

    @pl.when((p == 1) & (b < nb1))
    def _phase1():
        a = ab_ref[0]
        c = ab_ref[1]
        base = h_lo * crop_rows + b * out_rows_per_step
        xc = xc_ref[pl.ds(base, out_rows_per_step)]
        o_ref[...] = jnp.maximum(xc * a + c, 0.0)


@functools.partial(jax.jit, static_argnames=("stride", "padding"))
def _forward(x, w_t, gamma, beta, *, stride=1, padding=1):
    N, Cin, H, W = x.shape
    Cin_w, Cout, kH, kW = w_t.shape
    assert Cin == 1 and Cout == 1 and kH == 1 and kW == 1 and stride == 1

    Ho = (H - 1) * stride - 2 * padding + kH
    Wo = (W - 1) * stride - 2 * padding + kW
    assert Ho > 0 and Wo > 0
    assert N % LANE == 0 and (W * N) % LANE == 0 and (padding * N) % LANE == 0

    rows_per_slab = W * N // LANE
    crop_rows = Wo * N // LANE
    pad_rows = padding * N // LANE

    spb = 1
    for cand in (16, 13, 10, 8, 5, 4, 2):
        if H % cand == 0:
            spb = cand
            break
    nb0 = H // spb

    out_rows = Ho * crop_rows
    nb1 = 16
    while out_rows % nb1 != 0:
        nb1 //= 2
    out_rows_per_step = out_rows // nb1
    nsteps = max(nb0, nb1)

    z = jnp.transpose(x, (2, 3, 1, 0)).reshape(H * rows_per_slab, LANE)
    w1 = w_t.reshape(1).astype(jnp.float32)
    gamma32 = gamma.astype(jnp.float32)
    beta32 = beta.astype(jnp.float32)
    inv_cnt = 1.0 / float(N * Ho * Wo)

    out = pl.pallas_call(
        functools.partial(
            _fused_kernel, spb=spb, nb0=nb0, nb1=nb1,
            rows_per_slab=rows_per_slab,
            crop_rows=crop_rows, pad_rows=pad_rows,
            h_lo=padding, h_hi=padding + Ho,
            out_rows_per_step=out_rows_per_step, inv_cnt=inv_cnt),
        out_shape=jax.ShapeDtypeStruct((out_rows, LANE), x.dtype),
        grid=(2, nsteps),
        in_specs=[
            pl.BlockSpec(
                (spb * rows_per_slab, LANE),
                lambda p, b: (jnp.where(p == 0, jnp.minimum(b, nb0 - 1),
                                        nb0 - 1), 0)),
            pl.BlockSpec(memory_space=pltpu.MemorySpace.SMEM),
            pl.BlockSpec(memory_space=pltpu.MemorySpace.SMEM),
            pl.BlockSpec(memory_space=pltpu.MemorySpace.SMEM),
        ],
        out_specs=pl.BlockSpec((out_rows_per_step, LANE),
                               lambda p, b: (jnp.minimum(b, nb1 - 1) * p, 0)),
        scratch_shapes=[
            pltpu.VMEM((H * crop_rows, LANE), jnp.float32),
            pltpu.VMEM((2, SUBLANE, LANE), jnp.float32),
            pltpu.SMEM((2,), jnp.float32),
        ],
        compiler_params=pltpu.CompilerParams(
            dimension_semantics=("arbitrary", "arbitrary"),
            vmem_limit_bytes=VMEM_LIMIT),
    )(z, w1, gamma32, beta32)

    out4 = out.reshape(Ho, Wo, Cout, N).transpose(3, 2, 0, 1)
    return jax_layout.with_layout_constraint(
        out4,
        jax_layout.Layout(major_to_minor=(2, 3, 1, 0), tiling=((1, LANE),)))


def kernel(x, w_t, gamma, beta):
    return _forward(x, w_t, gamma, beta, stride=1, padding=1)


# speedup vs baseline: 2.1844x; 1.0036x over previous
import functools

import jax
import jax.numpy as jnp
from jax.experimental import layout as jax_layout
from jax.experimental import pallas as pl
from jax.experimental.pallas import tpu as pltpu

BN_EPS = 1e-5
LANE = 128
SUBLANE = 8
VMEM_LIMIT = 60 * 1024 * 1024


def _fused_kernel(x_ref, w_ref, gamma_ref, beta_ref, o_ref,
                  xc_ref, acc_ref, ab_ref, *,
                  spb, nb0, nb1, rows_per_slab, crop_rows, pad_rows,
                  h_lo, h_hi, out_rows_per_step, inv_cnt):
    p = pl.program_id(0)
    b = pl.program_id(1)

    @pl.when((p == 0) & (b < nb0))
    def _phase0():
        @pl.when(b == 0)
        def _():
            acc_ref[...] = jnp.zeros_like(acc_ref)

        v = x_ref[...].reshape(spb, rows_per_slab, LANE)
        vc = v[:, pad_rows:pad_rows + crop_rows, :]
        xc_ref[pl.ds(b * spb * crop_rows, spb * crop_rows)] = (
            vc.reshape(spb * crop_rows, LANE))

        gh = b * spb + jax.lax.broadcasted_iota(jnp.int32, (spb, 1, 1), 0)
        mask = ((gh >= h_lo) & (gh < h_hi)).astype(jnp.float32)
        vm = vc * mask
        vm8 = vm.reshape(-1, SUBLANE, LANE)
        acc_ref[0] += jnp.sum(vm8, axis=0)
        acc_ref[1] += jnp.sum(vm8 * vm8, axis=0)

        @pl.when(b == nb0 - 1)
        def _finalize():
            s1 = jnp.sum(acc_ref[0])
            s2 = jnp.sum(acc_ref[1])
            w = w_ref[0]
            mean_t = w * s1 * inv_cnt
            ex2_t = w * w * s2 * inv_cnt
            var = jnp.maximum(ex2_t - mean_t * mean_t, 0.0)
            a = gamma_ref[0] * jax.lax.rsqrt(var + BN_EPS)
            ab_ref[0] = w * a
            ab_ref[1] = beta_ref[0] - mean_t * a

    @pl.when((p == 1) & (b < nb1))
    def _phase1():
        a = ab_ref[0]
        c = ab_ref[1]
        base = h_lo * crop_rows + b * out_rows_per_step
        xc = xc_ref[pl.ds(base, out_rows_per_step)]
        o_ref[...] = jnp.maximum(xc * a + c, 0.0)


def _forward(x, w_t, gamma, beta, *, stride=1, padding=1):
    N, Cin, H, W = x.shape
    Cin_w, Cout, kH, kW = w_t.shape
    assert Cin == 1 and Cout == 1 and kH == 1 and kW == 1 and stride == 1

    Ho = (H - 1) * stride - 2 * padding + kH
    Wo = (W - 1) * stride - 2 * padding + kW
    assert Ho > 0 and Wo > 0
    assert N % LANE == 0 and (W * N) % LANE == 0 and (padding * N) % LANE == 0

    rows_per_slab = W * N // LANE
    crop_rows = Wo * N // LANE
    pad_rows = padding * N // LANE

    spb = 1
    for cand in (16, 13, 10, 8, 5, 4, 2):
        if H % cand == 0:
            spb = cand
            break
    nb0 = H // spb

    out_rows = Ho * crop_rows
    nb1 = 16
    while out_rows % nb1 != 0:
        nb1 //= 2
    out_rows_per_step = out_rows // nb1
    nsteps = max(nb0, nb1)

    z = jnp.transpose(x, (2, 3, 1, 0)).reshape(H * rows_per_slab, LANE)
    w1 = w_t.reshape(1).astype(jnp.float32)
    gamma32 = gamma.astype(jnp.float32)
    beta32 = beta.astype(jnp.float32)
    inv_cnt = 1.0 / float(N * Ho * Wo)

    out = pl.pallas_call(
        functools.partial(
            _fused_kernel, spb=spb, nb0=nb0, nb1=nb1,
            rows_per_slab=rows_per_slab,
            crop_rows=crop_rows, pad_rows=pad_rows,
            h_lo=padding, h_hi=padding + Ho,
            out_rows_per_step=out_rows_per_step, inv_cnt=inv_cnt),
        out_shape=jax.ShapeDtypeStruct((out_rows, LANE), x.dtype),
        grid=(2, nsteps),
        in_specs=[
            pl.BlockSpec(
                (spb * rows_per_slab, LANE),
                lambda p, b: (jnp.where(p == 0, jnp.minimum(b, nb0 - 1),
                                        nb0 - 1), 0)),
            pl.BlockSpec(memory_space=pltpu.MemorySpace.SMEM),
            pl.BlockSpec(memory_space=pltpu.MemorySpace.SMEM),
            pl.BlockSpec(memory_space=pltpu.MemorySpace.SMEM),
        ],
        out_specs=pl.BlockSpec((out_rows_per_step, LANE),
                               lambda p, b: (jnp.minimum(b, nb1 - 1) * p, 0)),
        scratch_shapes=[
            pltpu.VMEM((H * crop_rows, LANE), jnp.float32),
            pltpu.VMEM((2, SUBLANE, LANE), jnp.float32),
            pltpu.SMEM((2,), jnp.float32),
        ],
        compiler_params=pltpu.CompilerParams(
            dimension_semantics=("arbitrary", "arbitrary"),
            vmem_limit_bytes=VMEM_LIMIT),
    )(z, w1, gamma32, beta32)

    out4 = out.reshape(Ho, Wo, Cout, N).transpose(3, 2, 0, 1)
    return jax_layout.with_layout_constraint(
        out4,
        jax_layout.Layout(major_to_minor=(2, 3, 1, 0), tiling=((1, LANE),)))


@functools.lru_cache(maxsize=1)
def _jitted_forward():
    fmt = jax_layout.Format(
        jax_layout.Layout(major_to_minor=(2, 3, 1, 0), tiling=((1, LANE),)),
        jax.sharding.SingleDeviceSharding(jax.devices()[0]))
    return jax.jit(functools.partial(_forward, stride=1, padding=1),
                   out_shardings=fmt)


def kernel(x, w_t, gamma, beta):
    return _jitted_forward()(x, w_t, gamma, beta)
